# baseline (speedup 1.0000x reference)
import jax
import jax.numpy as jnp
from jax.experimental import pallas as pl

_N = 100000
_C = 80
_LOSS_WEIGHT = 1.0
_BC = 8


def _slab_loss(x, t, s, c0):
    e = jnp.exp(-jnp.abs(x))
    den = 1.0 + e
    sp = jnp.maximum(x, 0.0) + jnp.log(den)
    sig = jnp.where(x >= 0.0, 1.0, e) * pl.reciprocal(den, approx=True)
    neg = sp * sig * sig
    d = s - sig
    pos = (sp - x * s) * (d * d)
    c = jax.lax.broadcasted_iota(jnp.int32, x.shape, 0) + c0
    return jnp.sum(jnp.where(t == c, pos, neg))


def _qfl_kernel(x_ref, t_ref, s_ref, out_ref):
    i = pl.program_id(0)
    t = t_ref[...].reshape(1, _N)
    s = s_ref[...].reshape(1, _N)
    acc = _slab_loss(x_ref[...], t, s, i * 16)
    out_ref[...] = acc.reshape(1, 1, 1)


def kernel(inputs, targets, scores):
    x_t = inputs.T
    nb = _C // 16
    out = pl.pallas_call(
        _qfl_kernel,
        grid=(nb,),
        in_specs=[
            pl.BlockSpec((16, _N), lambda i: (i, 0)),
            pl.BlockSpec((_N,), lambda i: (0,)),
            pl.BlockSpec((_N,), lambda i: (0,)),
        ],
        out_specs=pl.BlockSpec((1, 1, 1), lambda i: (i, 0, 0)),
        out_shape=jax.ShapeDtypeStruct((nb, 1, 1), jnp.float32),
    )(x_t, targets.astype(jnp.int32), scores)
    return _LOSS_WEIGHT * (jnp.sum(out) / _N)


# speedup vs baseline: 1.0318x; 1.0318x over previous
import jax
import jax.numpy as jnp
from jax.experimental import pallas as pl

_N = 100000
_C = 80
_LOSS_WEIGHT = 1.0
_BC = 8
_STREAMS = 5
_NB = _C // _BC // _STREAMS


def _slab_loss(x, t, s, c0):
    e = jnp.exp(-jnp.abs(x))
    den = 1.0 + e
    sp = jnp.maximum(x, 0.0) + jnp.log(den)
    sig = jnp.where(x >= 0.0, 1.0, e) * pl.reciprocal(den, approx=True)
    neg = sp * sig * sig
    d = s - sig
    pos = (sp - x * s) * (d * d)
    c = jax.lax.broadcasted_iota(jnp.int32, x.shape, 0) + c0
    return jnp.sum(jnp.where(t == c, pos, neg))


def _qfl_kernel(*refs):
    x_refs, (t_ref, s_ref, out_ref) = refs[:_STREAMS], refs[_STREAMS:]
    i = pl.program_id(0)
    t = t_ref[...].reshape(1, _N)
    s = s_ref[...].reshape(1, _N)
    acc = _slab_loss(x_refs[0][...], t, s, (_NB * 0 + i) * _BC)
    for j in range(1, _STREAMS):
        acc += _slab_loss(x_refs[j][...], t, s, (_NB * j + i) * _BC)
    out_ref[...] = acc.reshape(1, 1, 1)


def _x_spec(j):
    return pl.BlockSpec((_BC, _N), lambda i, j=j: (_NB * j + i, 0))


def kernel(inputs, targets, scores):
    x_t = inputs.T
    out = pl.pallas_call(
        _qfl_kernel,
        grid=(_NB,),
        in_specs=[_x_spec(j) for j in range(_STREAMS)] + [
            pl.BlockSpec((_N,), lambda i: (0,)),
            pl.BlockSpec((_N,), lambda i: (0,)),
        ],
        out_specs=pl.BlockSpec((1, 1, 1), lambda i: (i, 0, 0)),
        out_shape=jax.ShapeDtypeStruct((_NB, 1, 1), jnp.float32),
    )(*([x_t] * _STREAMS), targets.astype(jnp.int32), scores)
    return _LOSS_WEIGHT * (jnp.sum(out) / _N)


# speedup vs baseline: 1.6330x; 1.5827x over previous
import jax
import jax.numpy as jnp
from jax.experimental import pallas as pl

_N = 100000
_C = 80
_LOSS_WEIGHT = 1.0
_BC = 8
_CH = 2048
_FULL = (_N // _CH) * _CH
_TAIL = _N - _FULL


def _chunk_val(x, t, s, ci):
    xm = jnp.maximum(x, -30.0)
    e = jnp.exp(-xm)
    den = 1.0 + e
    sig = pl.reciprocal(den, approx=True)
    sp = xm + jnp.log(den)
    neg = sp * sig * sig
    d = s - sig
    pos = (sp - x * s) * (d * d)
    return jnp.where(t == ci, pos, neg)


def _slab_loss(x_ref, t_ref, s_ref, c0):
    ci = jax.lax.broadcasted_iota(jnp.int32, (_BC, _CH), 0) + c0
    acc = jnp.zeros((_BC, _CH), jnp.float32)
    for k in range(0, _FULL, _CH):
        acc = acc + _chunk_val(
            x_ref[:, k:k + _CH],
            t_ref[k:k + _CH].reshape(1, _CH),
            s_ref[k:k + _CH].reshape(1, _CH),
            ci,
        )
    ci_t = jax.lax.broadcasted_iota(jnp.int32, (_BC, _TAIL), 0) + c0
    tail = _chunk_val(
        x_ref[:, _FULL:],
        t_ref[_FULL:].reshape(1, _TAIL),
        s_ref[_FULL:].reshape(1, _TAIL),
        ci_t,
    )
    return jnp.sum(acc) + jnp.sum(tail)


def _qfl_kernel(xa_ref, xb_ref, t_ref, s_ref, out_ref):
    i = pl.program_id(0)
    acc = _slab_loss(xa_ref, t_ref, s_ref, i * _BC)
    acc += _slab_loss(xb_ref, t_ref, s_ref, _C // 2 + i * _BC)
    out_ref[...] = acc.reshape(1, 1, 1)


def kernel(inputs, targets, scores):
    x_t = inputs.T
    nb = _C // _BC // 2
    out = pl.pallas_call(
        _qfl_kernel,
        grid=(nb,),
        in_specs=[
            pl.BlockSpec((_BC, _N), lambda i: (i, 0)),
            pl.BlockSpec((_BC, _N), lambda i: (i + _C // _BC // 2, 0)),
            pl.BlockSpec((_N,), lambda i: (0,)),
            pl.BlockSpec((_N,), lambda i: (0,)),
        ],
        out_specs=pl.BlockSpec((1, 1, 1), lambda i: (i, 0, 0)),
        out_shape=jax.ShapeDtypeStruct((nb, 1, 1), jnp.float32),
    )(x_t, x_t, targets.astype(jnp.int32), scores)
    return _LOSS_WEIGHT * (jnp.sum(out) / _N)


# speedup vs baseline: 1.7995x; 1.1020x over previous
import jax
import jax.numpy as jnp
from jax.experimental import pallas as pl

_N = 100000
_C = 80
_LOSS_WEIGHT = 1.0
_BC = 8
_CH = 1024
_FULL = (_N // _CH) * _CH
_TAIL = _N - _FULL


def _chunk_val(x, t, s, ci):
    xm = jnp.maximum(x, -30.0)
    e = jnp.exp(-xm)
    den = 1.0 + e
    sig = pl.reciprocal(den, approx=True)
    sp = xm + jnp.log(den)
    neg = sp * sig * sig
    d = s - sig
    pos = (sp - x * s) * (d * d)
    return jnp.where(t == ci, pos, neg)


def _slab_loss(x_ref, t_ref, s_ref, c0):
    ci = jax.lax.broadcasted_iota(jnp.int32, (_BC, _CH), 0) + c0
    acc = jnp.zeros((_BC, _CH), jnp.float32)
    for k in range(0, _FULL, _CH):
        acc = acc + _chunk_val(
            x_ref[:, k:k + _CH],
            t_ref[k:k + _CH].reshape(1, _CH),
            s_ref[k:k + _CH].reshape(1, _CH),
            ci,
        )
    ci_t = jax.lax.broadcasted_iota(jnp.int32, (_BC, _TAIL), 0) + c0
    tail = _chunk_val(
        x_ref[:, _FULL:],
        t_ref[_FULL:].reshape(1, _TAIL),
        s_ref[_FULL:].reshape(1, _TAIL),
        ci_t,
    )
    return jnp.sum(acc) + jnp.sum(tail)


def _qfl_kernel(xa_ref, xb_ref, t_ref, s_ref, out_ref):
    i = pl.program_id(0)
    acc = _slab_loss(xa_ref, t_ref, s_ref, i * _BC)
    acc += _slab_loss(xb_ref, t_ref, s_ref, _C // 2 + i * _BC)
    out_ref[...] = acc.reshape(1, 1, 1)


def kernel(inputs, targets, scores):
    x_t = inputs.T
    nb = _C // _BC // 2
    out = pl.pallas_call(
        _qfl_kernel,
        grid=(nb,),
        in_specs=[
            pl.BlockSpec((_BC, _N), lambda i: (i, 0)),
            pl.BlockSpec((_BC, _N), lambda i: (i + _C // _BC // 2, 0)),
            pl.BlockSpec((_N,), lambda i: (0,)),
            pl.BlockSpec((_N,), lambda i: (0,)),
        ],
        out_specs=pl.BlockSpec((1, 1, 1), lambda i: (i, 0, 0)),
        out_shape=jax.ShapeDtypeStruct((nb, 1, 1), jnp.float32),
    )(x_t, x_t, targets.astype(jnp.int32), scores)
    return _LOSS_WEIGHT * (jnp.sum(out) / _N)


# speedup vs baseline: 1.8059x; 1.0036x over previous
import jax
import jax.numpy as jnp
from jax.experimental import pallas as pl

_N = 100000
_C = 80
_LOSS_WEIGHT = 1.0
_BC = 8
_CH = 512
_FULL = (_N // _CH) * _CH
_TAIL = _N - _FULL


def _chunk_val(x, t, s, ci):
    xm = jnp.maximum(x, -30.0)
    e = jnp.exp(-xm)
    den = 1.0 + e
    sig = pl.reciprocal(den, approx=True)
    sp = xm + jnp.log(den)
    neg = sp * sig * sig
    d = s - sig
    pos = (sp - x * s) * (d * d)
    return jnp.where(t == ci, pos, neg)


def _slab_loss(x_ref, t_ref, s_ref, c0):
    ci = jax.lax.broadcasted_iota(jnp.int32, (_BC, _CH), 0) + c0
    acc = jnp.zeros((_BC, _CH), jnp.float32)
    for k in range(0, _FULL, _CH):
        acc = acc + _chunk_val(
            x_ref[:, k:k + _CH],
            t_ref[k:k + _CH].reshape(1, _CH),
            s_ref[k:k + _CH].reshape(1, _CH),
            ci,
        )
    ci_t = jax.lax.broadcasted_iota(jnp.int32, (_BC, _TAIL), 0) + c0
    tail = _chunk_val(
        x_ref[:, _FULL:],
        t_ref[_FULL:].reshape(1, _TAIL),
        s_ref[_FULL:].reshape(1, _TAIL),
        ci_t,
    )
    return jnp.sum(acc) + jnp.sum(tail)


def _qfl_kernel(xa_ref, xb_ref, t_ref, s_ref, out_ref):
    i = pl.program_id(0)
    acc = _slab_loss(xa_ref, t_ref, s_ref, i * _BC)
    acc += _slab_loss(xb_ref, t_ref, s_ref, _C // 2 + i * _BC)
    out_ref[...] = acc.reshape(1, 1, 1)


def kernel(inputs, targets, scores):
    x_t = inputs.T
    nb = _C // _BC // 2
    out = pl.pallas_call(
        _qfl_kernel,
        grid=(nb,),
        in_specs=[
            pl.BlockSpec((_BC, _N), lambda i: (i, 0)),
            pl.BlockSpec((_BC, _N), lambda i: (i + _C // _BC // 2, 0)),
            pl.BlockSpec((_N,), lambda i: (0,)),
            pl.BlockSpec((_N,), lambda i: (0,)),
        ],
        out_specs=pl.BlockSpec((1, 1, 1), lambda i: (i, 0, 0)),
        out_shape=jax.ShapeDtypeStruct((nb, 1, 1), jnp.float32),
    )(x_t, x_t, targets.astype(jnp.int32), scores)
    return _LOSS_WEIGHT * (jnp.sum(out) / _N)


# speedup vs baseline: 1.8358x; 1.0166x over previous
import jax
import jax.numpy as jnp
from jax.experimental import pallas as pl

_N = 100000
_C = 80
_LOSS_WEIGHT = 1.0
_BC = 8
_CH = 512
_FULL = (_N // _CH) * _CH
_TAIL = _N - _FULL


def _chunk_val(x, t, s, ci):
    xm = jnp.maximum(x, -30.0)
    e = jnp.exp2(xm * -1.4426950408889634)
    den = 1.0 + e
    sig = pl.reciprocal(den, approx=True)
    sp = xm + jnp.log2(den) * 0.6931471805599453
    hit = t == ci
    a = jnp.where(hit, sp - x * s, sp)
    b = jnp.where(hit, s - sig, sig)
    return a * (b * b)


def _slab_loss(x_ref, t_ref, s_ref, c0):
    ci = jax.lax.broadcasted_iota(jnp.int32, (_BC, _CH), 0) + c0
    acc = jnp.zeros((_BC, _CH), jnp.float32)
    for k in range(0, _FULL, _CH):
        acc = acc + _chunk_val(
            x_ref[:, k:k + _CH],
            t_ref[k:k + _CH].reshape(1, _CH),
            s_ref[k:k + _CH].reshape(1, _CH),
            ci,
        )
    ci_t = jax.lax.broadcasted_iota(jnp.int32, (_BC, _TAIL), 0) + c0
    tail = _chunk_val(
        x_ref[:, _FULL:],
        t_ref[_FULL:].reshape(1, _TAIL),
        s_ref[_FULL:].reshape(1, _TAIL),
        ci_t,
    )
    return jnp.sum(acc) + jnp.sum(tail)


def _qfl_kernel(xa_ref, xb_ref, t_ref, s_ref, out_ref):
    i = pl.program_id(0)
    acc = _slab_loss(xa_ref, t_ref, s_ref, i * _BC)
    acc += _slab_loss(xb_ref, t_ref, s_ref, _C // 2 + i * _BC)
    out_ref[...] = acc.reshape(1, 1, 1)


def kernel(inputs, targets, scores):
    x_t = inputs.T
    nb = _C // _BC // 2
    out = pl.pallas_call(
        _qfl_kernel,
        grid=(nb,),
        in_specs=[
            pl.BlockSpec((_BC, _N), lambda i: (i, 0)),
            pl.BlockSpec((_BC, _N), lambda i: (i + _C // _BC // 2, 0)),
            pl.BlockSpec((_N,), lambda i: (0,)),
            pl.BlockSpec((_N,), lambda i: (0,)),
        ],
        out_specs=pl.BlockSpec((1, 1, 1), lambda i: (i, 0, 0)),
        out_shape=jax.ShapeDtypeStruct((nb, 1, 1), jnp.float32),
    )(x_t, x_t, targets.astype(jnp.int32), scores)
    return _LOSS_WEIGHT * (jnp.sum(out) / _N)


# speedup vs baseline: 1.8368x; 1.0005x over previous
import jax
import jax.numpy as jnp
from jax.experimental import pallas as pl

_N = 100000
_C = 80
_LOSS_WEIGHT = 1.0
_BC = 8
_CH = 256
_FULL = (_N // _CH) * _CH
_TAIL = _N - _FULL


def _chunk_val(x, t, s, ci):
    xm = jnp.maximum(x, -30.0)
    e = jnp.exp2(xm * -1.4426950408889634)
    den = 1.0 + e
    sig = pl.reciprocal(den, approx=True)
    sp = xm + jnp.log2(den) * 0.6931471805599453
    hit = t == ci
    a = jnp.where(hit, sp - x * s, sp)
    b = jnp.where(hit, s - sig, sig)
    return a * (b * b)


def _slab_loss(x_ref, t_ref, s_ref, c0):
    ci = jax.lax.broadcasted_iota(jnp.int32, (_BC, _CH), 0) + c0
    acc = jnp.zeros((_BC, _CH), jnp.float32)
    for k in range(0, _FULL, _CH):
        acc = acc + _chunk_val(
            x_ref[:, k:k + _CH],
            t_ref[k:k + _CH].reshape(1, _CH),
            s_ref[k:k + _CH].reshape(1, _CH),
            ci,
        )
    ci_t = jax.lax.broadcasted_iota(jnp.int32, (_BC, _TAIL), 0) + c0
    tail = _chunk_val(
        x_ref[:, _FULL:],
        t_ref[_FULL:].reshape(1, _TAIL),
        s_ref[_FULL:].reshape(1, _TAIL),
        ci_t,
    )
    return jnp.sum(acc) + jnp.sum(tail)


def _qfl_kernel(xa_ref, xb_ref, t_ref, s_ref, out_ref):
    i = pl.program_id(0)
    acc = _slab_loss(xa_ref, t_ref, s_ref, i * _BC)
    acc += _slab_loss(xb_ref, t_ref, s_ref, _C // 2 + i * _BC)
    out_ref[...] = acc.reshape(1, 1, 1)


def kernel(inputs, targets, scores):
    x_t = inputs.T
    nb = _C // _BC // 2
    out = pl.pallas_call(
        _qfl_kernel,
        grid=(nb,),
        in_specs=[
            pl.BlockSpec((_BC, _N), lambda i: (i, 0)),
            pl.BlockSpec((_BC, _N), lambda i: (i + _C // _BC // 2, 0)),
            pl.BlockSpec((_N,), lambda i: (0,)),
            pl.BlockSpec((_N,), lambda i: (0,)),
        ],
        out_specs=pl.BlockSpec((1, 1, 1), lambda i: (i, 0, 0)),
        out_shape=jax.ShapeDtypeStruct((nb, 1, 1), jnp.float32),
    )(x_t, x_t, targets.astype(jnp.int32), scores)
    return _LOSS_WEIGHT * (jnp.sum(out) / _N)


# speedup vs baseline: 1.8389x; 1.0011x over previous
import jax
import jax.numpy as jnp
from jax.experimental import pallas as pl

_N = 100000
_C = 80
_LOSS_WEIGHT = 1.0
_BC = 8
_CH = 512
_FULL = (_N // _CH) * _CH
_TAIL = _N - _FULL


def _chunk_val(x, t, s, ci):
    xm = jnp.maximum(x, -30.0)
    e = jnp.exp2(xm * -1.4426950408889634)
    den = 1.0 + e
    sig = pl.reciprocal(den, approx=True)
    sp = xm + jnp.log2(den) * 0.6931471805599453
    hit = t == ci
    a = jnp.where(hit, sp - x * s, sp)
    b = jnp.where(hit, s - sig, sig)
    return a * (b * b)


def _slab_loss(x_ref, t_ref, s_ref, c0):
    ci = jax.lax.broadcasted_iota(jnp.int32, (_BC, _CH), 0) + c0
    acc = jnp.zeros((_BC, _CH), jnp.float32)
    for k in range(0, _FULL, _CH):
        acc = acc + _chunk_val(
            x_ref[:, k:k + _CH],
            t_ref[k:k + _CH].reshape(1, _CH),
            s_ref[k:k + _CH].reshape(1, _CH),
            ci,
        )
    ci_t = jax.lax.broadcasted_iota(jnp.int32, (_BC, _TAIL), 0) + c0
    tail = _chunk_val(
        x_ref[:, _FULL:],
        t_ref[_FULL:].reshape(1, _TAIL),
        s_ref[_FULL:].reshape(1, _TAIL),
        ci_t,
    )
    return jnp.sum(acc) + jnp.sum(tail)


def _qfl_kernel(xa_ref, xb_ref, t_ref, s_ref, out_ref):
    i = pl.program_id(0)
    acc = _slab_loss(xa_ref, t_ref, s_ref, i * _BC)
    acc += _slab_loss(xb_ref, t_ref, s_ref, _C // 2 + i * _BC)
    out_ref[...] = acc.reshape(1, 1, 1)


def kernel(inputs, targets, scores):
    x_t = inputs.T
    nb = _C // _BC // 2
    out = pl.pallas_call(
        _qfl_kernel,
        grid=(nb,),
        in_specs=[
            pl.BlockSpec((_BC, _N), lambda i: (i, 0)),
            pl.BlockSpec((_BC, _N), lambda i: (i + _C // _BC // 2, 0)),
            pl.BlockSpec((_N,), lambda i: (0,)),
            pl.BlockSpec((_N,), lambda i: (0,)),
        ],
        out_specs=pl.BlockSpec((1, 1, 1), lambda i: (i, 0, 0)),
        out_shape=jax.ShapeDtypeStruct((nb, 1, 1), jnp.float32),
    )(x_t, x_t, targets.astype(jnp.int32), scores)
    return _LOSS_WEIGHT * (jnp.sum(out) / _N)
